# Initial kernel scaffold; baseline (speedup 1.0000x reference)
import jax
import jax.numpy as jnp
from jax.experimental import pallas as pl


def kernel(h, X_0, X_1, rl_ij_0, rl_ij_1, t_ij, r_ij, Wq, Wk, Wre, Wrs, gv1, gv2, Wvq, Wvk0, Wvk1, gtW, gtb, lng, lnb, gwg, gwW, gwb, neighbor_index, neighbor_mask):
    raise NotImplementedError("write your pallas kernel here")


# speedup vs baseline: 2.0542x; 2.0542x over previous
import functools
import math

import jax
import jax.numpy as jnp
from jax import lax
from jax.experimental import pallas as pl
from jax.experimental.pallas import tpu as pltpu
from jax.experimental.pallas import tpu_sc as plsc

N, K, D, H = 10000, 16, 128, 8
DH = D // H
MULT = 5
CUTOFF = 5.0
E = N * K
NW = 32
PER_W = 5120
E_PAD = NW * PER_W

NB = 40
EB = NB * K
GRID = N // NB


def _sc_gather_call(table, idx_pad, ch):
    Dt = table.shape[1]
    n_ch = PER_W // ch
    mesh = plsc.VectorSubcoreMesh(core_axis_name="c", subcore_axis_name="s")

    @functools.partial(
        pl.kernel,
        mesh=mesh,
        out_type=jax.ShapeDtypeStruct((E_PAD, Dt), jnp.float32),
        scratch_types=[
            pltpu.VMEM((ch,), jnp.int32),
            pltpu.VMEM((ch, Dt), jnp.float32),
            pltpu.SemaphoreType.DMA,
        ],
    )
    def gather_k(table_hbm, idx_hbm, out_hbm, idx_v, rows_v, sem):
        wid = lax.axis_index("s") * 2 + lax.axis_index("c")
        base = wid * PER_W

        def body(j, carry):
            off = base + j * ch
            pltpu.sync_copy(idx_hbm.at[pl.ds(off, ch)], idx_v)
            pltpu.async_copy(table_hbm.at[idx_v], rows_v, sem).wait()
            pltpu.sync_copy(rows_v, out_hbm.at[pl.ds(off, ch)])
            return carry

        lax.fori_loop(0, n_ch, body, 0)

    return gather_k(table, idx_pad)


def _node_body(h_ref, Wq_ref, Wk_ref, gv1_ref, gv2_ref, lng_ref, lnb_ref,
               q_ref, kv_ref):
    h = h_ref[...]
    mu = jnp.mean(h, axis=-1, keepdims=True)
    var = jnp.mean((h - mu) * (h - mu), axis=-1, keepdims=True)
    hn = (h - mu) * lax.rsqrt(var + 1e-5) * lng_ref[...] + lnb_ref[...]
    q_ref[...] = jnp.dot(hn, Wq_ref[...], preferred_element_type=jnp.float32)
    kv_ref[:, :D] = jnp.dot(hn, Wk_ref[...], preferred_element_type=jnp.float32)
    g = jax.nn.silu(jnp.dot(hn, gv1_ref[...], preferred_element_type=jnp.float32))
    kv_ref[:, D:] = jnp.dot(g, gv2_ref[...], preferred_element_type=jnp.float32)


def _node_call(h, Wq, Wk, gv1, gv2, lng, lnb):
    BN = 1000
    w_spec = lambda shp: pl.BlockSpec(shp, lambda i: (0, 0))
    return pl.pallas_call(
        _node_body,
        grid=(N // BN,),
        in_specs=[
            pl.BlockSpec((BN, D), lambda i: (i, 0)),
            w_spec((D, D)), w_spec((D, D)), w_spec((D, D)),
            w_spec((D, MULT * D)), w_spec((1, D)), w_spec((1, D)),
        ],
        out_specs=[
            pl.BlockSpec((BN, D), lambda i: (i, 0)),
            pl.BlockSpec((BN, D + MULT * D), lambda i: (i, 0)),
        ],
        out_shape=[
            jax.ShapeDtypeStruct((N, D), jnp.float32),
            jax.ShapeDtypeStruct((N, D + MULT * D), jnp.float32),
        ],
    )(h, Wq, Wk, gv1, gv2, lng, lnb)


def _seg_sum(x):
    return jnp.sum(x.reshape(NB, K, D), axis=1)


def _to_edges(x):
    return jnp.broadcast_to(x[:, None, :], (NB, K, D)).reshape(EB, D)


def _edge_body(t_ref, r_ref, m_ref, rl0_ref, rl1_ref, q_ref, h_ref,
               x0_ref, x1_ref, kvj_ref, xj_ref,
               Wre_ref, Wrs_ref, gtW_ref, gtb_ref, Wvq_ref, Wvk0_ref, Wvk1_ref,
               hout_ref, x0out_ref, x1out_ref, qv0_ref, qv1_ref, kvt_ref,
               gt_ref):
    t = t_ref[...]
    re = jax.nn.silu(jnp.dot(t, Wre_ref[...], preferred_element_type=jnp.float32))
    rs = jnp.dot(t, Wrs_ref[...], preferred_element_type=jnp.float32)
    kj = kvj_ref[:, :D]
    qe = _to_edges(q_ref[...])
    p = qe * kj * re
    row = lax.broadcasted_iota(jnp.int32, (D, D), 0) // DH
    col = lax.broadcasted_iota(jnp.int32, (D, D), 1) // DH
    bd = (row == col).astype(jnp.float32)
    af_pre = jnp.dot(p, bd, preferred_element_type=jnp.float32)
    r = r_ref[...]
    cut = 0.5 * (jnp.cos(r * (math.pi / CUTOFF)) + 1.0)
    cut = cut * (r < CUTOFF).astype(jnp.float32)
    scal = cut * m_ref[...] * (1.0 / 16.0)
    af = af_pre * scal

    vj = kvj_ref[:, D:]
    c0 = vj[:, 0 * D:1 * D] * rs[:, 0 * D:1 * D] * af
    c1 = vj[:, 1 * D:2 * D] * rs[:, 1 * D:2 * D] * af
    c2 = vj[:, 2 * D:3 * D] * rs[:, 2 * D:3 * D] * af
    c3 = vj[:, 3 * D:4 * D] * rs[:, 3 * D:4 * D] * af
    c4 = vj[:, 4 * D:5 * D] * rs[:, 4 * D:5 * D] * af

    hout_ref[...] = h_ref[...] + _seg_sum(c0)

    rl0 = rl0_ref[...]
    rl1 = rl1_ref[...]
    Wvq = Wvq_ref[...]
    Wvk0 = Wvk0_ref[...]
    Wvk1 = Wvk1_ref[...]
    for c in range(3):
        sl = slice(c * D, (c + 1) * D)
        dx = _seg_sum(c1 * rl0[:, c:c + 1] + c2 * xj_ref[:, sl])
        xo = x0_ref[:, sl] + dx
        x0out_ref[:, sl] = xo
        qv0_ref[:, sl] = jnp.dot(xo, Wvq, preferred_element_type=jnp.float32)
        kvt_ref[:, sl] = jnp.dot(xo, Wvk0, preferred_element_type=jnp.float32)
    for c in range(5):
        sl = slice(c * D, (c + 1) * D)
        dx = _seg_sum(c3 * rl1[:, c:c + 1] + c4 * xj_ref[:, 3 * D + c * D:3 * D + (c + 1) * D])
        xo = x1_ref[:, sl] + dx
        x1out_ref[:, sl] = xo
        qv1_ref[:, sl] = jnp.dot(xo, Wvq, preferred_element_type=jnp.float32)
        kvt_ref[:, 3 * D + c * D:3 * D + (c + 1) * D] = jnp.dot(
            xo, Wvk1, preferred_element_type=jnp.float32)

    gt_ref[...] = jax.nn.silu(
        jnp.dot(t, gtW_ref[...], preferred_element_type=jnp.float32) + gtb_ref[...])


def _edge_call(t2, r2, m2, rl0f, rl1f, q_tab, h, x0f, x1f, kvj, xj,
               Wre, Wrs, gtW, gtb, Wvq, Wvk0, Wvk1):
    eb = lambda d: pl.BlockSpec((EB, d), lambda i: (i, 0))
    nb = lambda d: pl.BlockSpec((NB, d), lambda i: (i, 0))
    w_spec = lambda shp: pl.BlockSpec(shp, lambda i: (0, 0))
    return pl.pallas_call(
        _edge_body,
        grid=(GRID,),
        in_specs=[
            eb(D), eb(1), eb(1), eb(3), eb(5),
            nb(D), nb(D), nb(3 * D), nb(5 * D),
            eb(6 * D), eb(8 * D),
            w_spec((D, D)), w_spec((D, MULT * D)), w_spec((D, D)),
            w_spec((1, D)), w_spec((D, D)), w_spec((D, D)), w_spec((D, D)),
        ],
        out_specs=[
            nb(D), nb(3 * D), nb(5 * D), nb(3 * D), nb(5 * D), nb(8 * D),
            eb(D),
        ],
        out_shape=[
            jax.ShapeDtypeStruct((N, D), jnp.float32),
            jax.ShapeDtypeStruct((N, 3 * D), jnp.float32),
            jax.ShapeDtypeStruct((N, 5 * D), jnp.float32),
            jax.ShapeDtypeStruct((N, 3 * D), jnp.float32),
            jax.ShapeDtypeStruct((N, 5 * D), jnp.float32),
            jax.ShapeDtypeStruct((N, 8 * D), jnp.float32),
            jax.ShapeDtypeStruct((E, D), jnp.float32),
        ],
    )(t2, r2, m2, rl0f, rl1f, q_tab, h, x0f, x1f, kvj, xj,
      Wre, Wrs, gtW, gtb, Wvq, Wvk0, Wvk1)


def _final_body(t_ref, gt_ref, rl0_ref, rl1_ref, qv0_ref, qv1_ref, kv2_ref,
                gwg_ref, gwW_ref, gwb_ref, tout_ref):
    rl0 = rl0_ref[...]
    rl1 = rl1_ref[...]
    acc = jnp.zeros((EB, D), jnp.float32)
    for c in range(3):
        rep = kv2_ref[:, c * D:(c + 1) * D]
        rlc = rl0[:, c:c + 1]
        rej = rep - (rlc * rlc) * jnp.sum(rep, axis=1, keepdims=True)
        acc = acc + _to_edges(qv0_ref[:, c * D:(c + 1) * D]) * rej
    for c in range(5):
        rep = kv2_ref[:, 3 * D + c * D:3 * D + (c + 1) * D]
        rlc = rl1[:, c:c + 1]
        rej = rep - (rlc * rlc) * jnp.sum(rep, axis=1, keepdims=True)
        acc = acc + _to_edges(qv1_ref[:, c * D:(c + 1) * D]) * rej
    mu = jnp.mean(acc, axis=-1, keepdims=True)
    var = jnp.mean((acc - mu) * (acc - mu), axis=-1, keepdims=True)
    an = (acc - mu) * lax.rsqrt(var + 1e-5) * gwg_ref[...]
    w = jnp.dot(jax.nn.silu(an), gwW_ref[...],
                preferred_element_type=jnp.float32) + gwb_ref[...]
    tout_ref[...] = t_ref[...] + gt_ref[...] * w


def _final_call(t2, gt, rl0f, rl1f, qv0, qv1, kv2j, gwg, gwW, gwb):
    eb = lambda d: pl.BlockSpec((EB, d), lambda i: (i, 0))
    nb = lambda d: pl.BlockSpec((NB, d), lambda i: (i, 0))
    w_spec = lambda shp: pl.BlockSpec(shp, lambda i: (0, 0))
    return pl.pallas_call(
        _final_body,
        grid=(GRID,),
        in_specs=[
            eb(D), eb(D), eb(3), eb(5), nb(3 * D), nb(5 * D), eb(8 * D),
            w_spec((1, D)), w_spec((D, D)), w_spec((1, D)),
        ],
        out_specs=eb(D),
        out_shape=jax.ShapeDtypeStruct((E, D), jnp.float32),
    )(t2, gt, rl0f, rl1f, qv0, qv1, kv2j, gwg, gwW, gwb)


def kernel(h, X_0, X_1, rl_ij_0, rl_ij_1, t_ij, r_ij, Wq, Wk, Wre, Wrs,
           gv1, gv2, Wvq, Wvk0, Wvk1, gtW, gtb, lng, lnb, gwg, gwW, gwb,
           neighbor_index, neighbor_mask):
    row = lambda v: v.reshape(1, D)
    q_tab, kv_tab = _node_call(h, Wq, Wk, gv1, gv2, row(lng), row(lnb))

    idx = neighbor_index.reshape(E).astype(jnp.int32)
    idx_pad = jnp.concatenate([idx, jnp.zeros((E_PAD - E,), jnp.int32)])

    kvj = _sc_gather_call(kv_tab, idx_pad, ch=128)
    x_cat = jnp.concatenate([X_0.reshape(N, 3 * D), X_1.reshape(N, 5 * D)],
                            axis=1)
    xj = _sc_gather_call(x_cat, idx_pad, ch=64)

    t2 = t_ij.reshape(E, D)
    r2 = r_ij.reshape(E, 1)
    m2 = neighbor_mask.reshape(E, 1).astype(jnp.float32)
    rl0f = rl_ij_0.reshape(E, 3)
    rl1f = rl_ij_1.reshape(E, 5)

    (hout, x0out, x1out, qv0, qv1, kv2_tab, gt) = _edge_call(
        t2, r2, m2, rl0f, rl1f, q_tab, h,
        X_0.reshape(N, 3 * D), X_1.reshape(N, 5 * D), kvj, xj,
        Wre, Wrs, gtW, row(gtb), Wvq, Wvk0, Wvk1)

    kv2j = _sc_gather_call(kv2_tab, idx_pad, ch=64)

    tout = _final_call(t2, gt, rl0f, rl1f, qv0, qv1, kv2j,
                       row(gwg), gwW, row(gwb))

    return (hout, x0out.reshape(N, 3, D), x1out.reshape(N, 5, D),
            tout.reshape(N, K, D))


# speedup vs baseline: 2.2881x; 1.1138x over previous
import functools
import math

import jax
import jax.numpy as jnp
from jax import lax
from jax.experimental import pallas as pl
from jax.experimental.pallas import tpu as pltpu
from jax.experimental.pallas import tpu_sc as plsc

N, K, D, H = 10000, 16, 128, 8
DH = D // H
MULT = 5
CUTOFF = 5.0
E = N * K
NW = 32
PER_W = 5120
E_PAD = NW * PER_W

NB = 40
EB = NB * K
GRID = N // NB


def _sc_gather_call(table, idx_pad, ch):
    Dt = table.shape[1]
    n_ch = PER_W // ch
    assert n_ch % 2 == 0
    mesh = plsc.VectorSubcoreMesh(core_axis_name="c", subcore_axis_name="s")

    @functools.partial(
        pl.kernel,
        mesh=mesh,
        out_type=jax.ShapeDtypeStruct((E_PAD, Dt), jnp.float32),
        scratch_types=[
            pltpu.VMEM((PER_W,), jnp.int32),
            pltpu.VMEM((ch, Dt), jnp.float32),
            pltpu.VMEM((ch, Dt), jnp.float32),
            pltpu.SemaphoreType.DMA,
            pltpu.SemaphoreType.DMA,
            pltpu.SemaphoreType.DMA,
            pltpu.SemaphoreType.DMA,
        ],
    )
    def gather_k(table_hbm, idx_hbm, out_hbm, idx_v, rows0, rows1,
                 gsem0, gsem1, ssem0, ssem1):
        wid = lax.axis_index("s") * 2 + lax.axis_index("c")
        base = wid * PER_W
        pltpu.sync_copy(idx_hbm.at[pl.ds(base, PER_W)], idx_v)
        bufs = ((rows0, gsem0, ssem0), (rows1, gsem1, ssem1))

        def gstart(j, buf, gsem):
            pltpu.make_async_copy(
                table_hbm.at[idx_v.at[pl.ds(j * ch, ch)]], buf, gsem).start()

        gstart(0, rows0, gsem0)
        gstart(1, rows1, gsem1)

        def body(jj, carry):
            for b, (buf, gsem, ssem) in enumerate(bufs):
                j = 2 * jj + b
                pltpu.make_async_copy(
                    table_hbm.at[idx_v.at[pl.ds(j * ch, ch)]], buf,
                    gsem).wait()
                out_slice = out_hbm.at[pl.ds(base + j * ch, ch)]
                pltpu.make_async_copy(buf, out_slice, ssem).start()
                pltpu.make_async_copy(buf, out_slice, ssem).wait()

                @pl.when(j + 2 < n_ch)
                def _():
                    gstart(j + 2, buf, gsem)
            return carry

        lax.fori_loop(0, n_ch // 2, body, 0)

    return gather_k(table, idx_pad)


def _node_body(h_ref, Wq_ref, Wk_ref, gv1_ref, gv2_ref, lng_ref, lnb_ref,
               q_ref, kv_ref):
    h = h_ref[...]
    mu = jnp.mean(h, axis=-1, keepdims=True)
    var = jnp.mean((h - mu) * (h - mu), axis=-1, keepdims=True)
    hn = (h - mu) * lax.rsqrt(var + 1e-5) * lng_ref[...] + lnb_ref[...]
    q_ref[...] = jnp.dot(hn, Wq_ref[...], preferred_element_type=jnp.float32)
    kv_ref[:, :D] = jnp.dot(hn, Wk_ref[...], preferred_element_type=jnp.float32)
    g = jax.nn.silu(jnp.dot(hn, gv1_ref[...], preferred_element_type=jnp.float32))
    kv_ref[:, D:] = jnp.dot(g, gv2_ref[...], preferred_element_type=jnp.float32)


def _node_call(h, Wq, Wk, gv1, gv2, lng, lnb):
    BN = 1000
    w_spec = lambda shp: pl.BlockSpec(shp, lambda i: (0, 0))
    return pl.pallas_call(
        _node_body,
        grid=(N // BN,),
        in_specs=[
            pl.BlockSpec((BN, D), lambda i: (i, 0)),
            w_spec((D, D)), w_spec((D, D)), w_spec((D, D)),
            w_spec((D, MULT * D)), w_spec((1, D)), w_spec((1, D)),
        ],
        out_specs=[
            pl.BlockSpec((BN, D), lambda i: (i, 0)),
            pl.BlockSpec((BN, D + MULT * D), lambda i: (i, 0)),
        ],
        out_shape=[
            jax.ShapeDtypeStruct((N, D), jnp.float32),
            jax.ShapeDtypeStruct((N, D + MULT * D), jnp.float32),
        ],
    )(h, Wq, Wk, gv1, gv2, lng, lnb)


def _seg_sum(x):
    return jnp.sum(x.reshape(NB, K, D), axis=1)


def _to_edges(x):
    return jnp.broadcast_to(x[:, None, :], (NB, K, D)).reshape(EB, D)


def _edge_body(t_ref, r_ref, m_ref, rl0_ref, rl1_ref, q_ref, h_ref,
               x0_ref, x1_ref, kvj_ref, xj_ref,
               Wre_ref, Wrs_ref, gtW_ref, gtb_ref, Wvq_ref, Wvk0_ref, Wvk1_ref,
               hout_ref, x0out_ref, x1out_ref, qv0_ref, qv1_ref, kvt_ref,
               gt_ref):
    t = t_ref[...]
    re = jax.nn.silu(jnp.dot(t, Wre_ref[...], preferred_element_type=jnp.float32))
    rs = jnp.dot(t, Wrs_ref[...], preferred_element_type=jnp.float32)
    kj = kvj_ref[:, :D]
    qe = _to_edges(q_ref[...])
    p = qe * kj * re
    row = lax.broadcasted_iota(jnp.int32, (D, D), 0) // DH
    col = lax.broadcasted_iota(jnp.int32, (D, D), 1) // DH
    bd = (row == col).astype(jnp.float32)
    af_pre = jnp.dot(p, bd, preferred_element_type=jnp.float32)
    r = r_ref[...]
    cut = 0.5 * (jnp.cos(r * (math.pi / CUTOFF)) + 1.0)
    cut = cut * (r < CUTOFF).astype(jnp.float32)
    scal = cut * m_ref[...] * (1.0 / 16.0)
    af = af_pre * scal

    vj = kvj_ref[:, D:]
    c0 = vj[:, 0 * D:1 * D] * rs[:, 0 * D:1 * D] * af
    c1 = vj[:, 1 * D:2 * D] * rs[:, 1 * D:2 * D] * af
    c2 = vj[:, 2 * D:3 * D] * rs[:, 2 * D:3 * D] * af
    c3 = vj[:, 3 * D:4 * D] * rs[:, 3 * D:4 * D] * af
    c4 = vj[:, 4 * D:5 * D] * rs[:, 4 * D:5 * D] * af

    hout_ref[...] = h_ref[...] + _seg_sum(c0)

    rl0 = rl0_ref[...]
    rl1 = rl1_ref[...]
    Wvq = Wvq_ref[...]
    Wvk0 = Wvk0_ref[...]
    Wvk1 = Wvk1_ref[...]
    for c in range(3):
        sl = slice(c * D, (c + 1) * D)
        dx = _seg_sum(c1 * rl0[:, c:c + 1] + c2 * xj_ref[:, sl])
        xo = x0_ref[:, sl] + dx
        x0out_ref[:, sl] = xo
        qv0_ref[:, sl] = jnp.dot(xo, Wvq, preferred_element_type=jnp.float32)
        kvt_ref[:, sl] = jnp.dot(xo, Wvk0, preferred_element_type=jnp.float32)
    for c in range(5):
        sl = slice(c * D, (c + 1) * D)
        dx = _seg_sum(c3 * rl1[:, c:c + 1] + c4 * xj_ref[:, 3 * D + c * D:3 * D + (c + 1) * D])
        xo = x1_ref[:, sl] + dx
        x1out_ref[:, sl] = xo
        qv1_ref[:, sl] = jnp.dot(xo, Wvq, preferred_element_type=jnp.float32)
        kvt_ref[:, 3 * D + c * D:3 * D + (c + 1) * D] = jnp.dot(
            xo, Wvk1, preferred_element_type=jnp.float32)

    gt_ref[...] = jax.nn.silu(
        jnp.dot(t, gtW_ref[...], preferred_element_type=jnp.float32) + gtb_ref[...])


def _edge_call(t2, r2, m2, rl0f, rl1f, q_tab, h, x0f, x1f, kvj, xj,
               Wre, Wrs, gtW, gtb, Wvq, Wvk0, Wvk1):
    eb = lambda d: pl.BlockSpec((EB, d), lambda i: (i, 0))
    nb = lambda d: pl.BlockSpec((NB, d), lambda i: (i, 0))
    w_spec = lambda shp: pl.BlockSpec(shp, lambda i: (0, 0))
    return pl.pallas_call(
        _edge_body,
        grid=(GRID,),
        in_specs=[
            eb(D), eb(1), eb(1), eb(3), eb(5),
            nb(D), nb(D), nb(3 * D), nb(5 * D),
            eb(6 * D), eb(8 * D),
            w_spec((D, D)), w_spec((D, MULT * D)), w_spec((D, D)),
            w_spec((1, D)), w_spec((D, D)), w_spec((D, D)), w_spec((D, D)),
        ],
        out_specs=[
            nb(D), nb(3 * D), nb(5 * D), nb(3 * D), nb(5 * D), nb(8 * D),
            eb(D),
        ],
        out_shape=[
            jax.ShapeDtypeStruct((N, D), jnp.float32),
            jax.ShapeDtypeStruct((N, 3 * D), jnp.float32),
            jax.ShapeDtypeStruct((N, 5 * D), jnp.float32),
            jax.ShapeDtypeStruct((N, 3 * D), jnp.float32),
            jax.ShapeDtypeStruct((N, 5 * D), jnp.float32),
            jax.ShapeDtypeStruct((N, 8 * D), jnp.float32),
            jax.ShapeDtypeStruct((E, D), jnp.float32),
        ],
    )(t2, r2, m2, rl0f, rl1f, q_tab, h, x0f, x1f, kvj, xj,
      Wre, Wrs, gtW, gtb, Wvq, Wvk0, Wvk1)


def _final_body(t_ref, gt_ref, rl0_ref, rl1_ref, qv0_ref, qv1_ref, kv2_ref,
                gwg_ref, gwW_ref, gwb_ref, tout_ref):
    rl0 = rl0_ref[...]
    rl1 = rl1_ref[...]
    acc = jnp.zeros((EB, D), jnp.float32)
    for c in range(3):
        rep = kv2_ref[:, c * D:(c + 1) * D]
        rlc = rl0[:, c:c + 1]
        rej = rep - (rlc * rlc) * jnp.sum(rep, axis=1, keepdims=True)
        acc = acc + _to_edges(qv0_ref[:, c * D:(c + 1) * D]) * rej
    for c in range(5):
        rep = kv2_ref[:, 3 * D + c * D:3 * D + (c + 1) * D]
        rlc = rl1[:, c:c + 1]
        rej = rep - (rlc * rlc) * jnp.sum(rep, axis=1, keepdims=True)
        acc = acc + _to_edges(qv1_ref[:, c * D:(c + 1) * D]) * rej
    mu = jnp.mean(acc, axis=-1, keepdims=True)
    var = jnp.mean((acc - mu) * (acc - mu), axis=-1, keepdims=True)
    an = (acc - mu) * lax.rsqrt(var + 1e-5) * gwg_ref[...]
    w = jnp.dot(jax.nn.silu(an), gwW_ref[...],
                preferred_element_type=jnp.float32) + gwb_ref[...]
    tout_ref[...] = t_ref[...] + gt_ref[...] * w


def _final_call(t2, gt, rl0f, rl1f, qv0, qv1, kv2j, gwg, gwW, gwb):
    eb = lambda d: pl.BlockSpec((EB, d), lambda i: (i, 0))
    nb = lambda d: pl.BlockSpec((NB, d), lambda i: (i, 0))
    w_spec = lambda shp: pl.BlockSpec(shp, lambda i: (0, 0))
    return pl.pallas_call(
        _final_body,
        grid=(GRID,),
        in_specs=[
            eb(D), eb(D), eb(3), eb(5), nb(3 * D), nb(5 * D), eb(8 * D),
            w_spec((1, D)), w_spec((D, D)), w_spec((1, D)),
        ],
        out_specs=eb(D),
        out_shape=jax.ShapeDtypeStruct((E, D), jnp.float32),
    )(t2, gt, rl0f, rl1f, qv0, qv1, kv2j, gwg, gwW, gwb)


def kernel(h, X_0, X_1, rl_ij_0, rl_ij_1, t_ij, r_ij, Wq, Wk, Wre, Wrs,
           gv1, gv2, Wvq, Wvk0, Wvk1, gtW, gtb, lng, lnb, gwg, gwW, gwb,
           neighbor_index, neighbor_mask):
    row = lambda v: v.reshape(1, D)
    q_tab, kv_tab = _node_call(h, Wq, Wk, gv1, gv2, row(lng), row(lnb))

    idx = neighbor_index.reshape(E).astype(jnp.int32)
    idx_pad = jnp.concatenate([idx, jnp.zeros((E_PAD - E,), jnp.int32)])

    kvj = _sc_gather_call(kv_tab, idx_pad, ch=64)
    x_cat = jnp.concatenate([X_0.reshape(N, 3 * D), X_1.reshape(N, 5 * D)],
                            axis=1)
    xj = _sc_gather_call(x_cat, idx_pad, ch=40)

    t2 = t_ij.reshape(E, D)
    r2 = r_ij.reshape(E, 1)
    m2 = neighbor_mask.reshape(E, 1).astype(jnp.float32)
    rl0f = rl_ij_0.reshape(E, 3)
    rl1f = rl_ij_1.reshape(E, 5)

    (hout, x0out, x1out, qv0, qv1, kv2_tab, gt) = _edge_call(
        t2, r2, m2, rl0f, rl1f, q_tab, h,
        X_0.reshape(N, 3 * D), X_1.reshape(N, 5 * D), kvj, xj,
        Wre, Wrs, gtW, row(gtb), Wvq, Wvk0, Wvk1)

    kv2j = _sc_gather_call(kv2_tab, idx_pad, ch=40)

    tout = _final_call(t2, gt, rl0f, rl1f, qv0, qv1, kv2j,
                       row(gwg), gwW, row(gwb))

    return (hout, x0out.reshape(N, 3, D), x1out.reshape(N, 5, D),
            tout.reshape(N, K, D))


# speedup vs baseline: 2.4147x; 1.0553x over previous
import functools
import math

import jax
import jax.numpy as jnp
from jax import lax
from jax.experimental import pallas as pl
from jax.experimental.pallas import tpu as pltpu
from jax.experimental.pallas import tpu_sc as plsc

N, K, D, H = 10000, 16, 128, 8
DH = D // H
MULT = 5
CUTOFF = 5.0
E = N * K
NW = 32
PER_W = 5120
E_PAD = NW * PER_W

NB = 80
EB = NB * K
GRID = N // NB


def _sc_gather_call(table, idx_pad, ch):
    Dt = table.shape[1]
    n_ch = PER_W // ch
    assert n_ch % 2 == 0
    mesh = plsc.VectorSubcoreMesh(core_axis_name="c", subcore_axis_name="s")

    @functools.partial(
        pl.kernel,
        mesh=mesh,
        out_type=jax.ShapeDtypeStruct((E_PAD, Dt), jnp.float32),
        scratch_types=[
            pltpu.VMEM((PER_W,), jnp.int32),
            pltpu.VMEM((ch, Dt), jnp.float32),
            pltpu.VMEM((ch, Dt), jnp.float32),
            pltpu.SemaphoreType.DMA,
            pltpu.SemaphoreType.DMA,
            pltpu.SemaphoreType.DMA,
            pltpu.SemaphoreType.DMA,
        ],
    )
    def gather_k(table_hbm, idx_hbm, out_hbm, idx_v, rows0, rows1,
                 gsem0, gsem1, ssem0, ssem1):
        wid = lax.axis_index("s") * 2 + lax.axis_index("c")
        base = wid * PER_W
        pltpu.sync_copy(idx_hbm.at[pl.ds(base, PER_W)], idx_v)
        bufs = ((rows0, gsem0, ssem0), (rows1, gsem1, ssem1))

        def gstart(j, buf, gsem):
            pltpu.make_async_copy(
                table_hbm.at[idx_v.at[pl.ds(j * ch, ch)]], buf, gsem).start()

        gstart(0, rows0, gsem0)
        gstart(1, rows1, gsem1)

        def body(jj, carry):
            for b, (buf, gsem, ssem) in enumerate(bufs):
                j = 2 * jj + b
                pltpu.make_async_copy(
                    table_hbm.at[idx_v.at[pl.ds(j * ch, ch)]], buf,
                    gsem).wait()
                out_slice = out_hbm.at[pl.ds(base + j * ch, ch)]
                pltpu.make_async_copy(buf, out_slice, ssem).start()
                pltpu.make_async_copy(buf, out_slice, ssem).wait()

                @pl.when(j + 2 < n_ch)
                def _():
                    gstart(j + 2, buf, gsem)
            return carry

        lax.fori_loop(0, n_ch // 2, body, 0)

    return gather_k(table, idx_pad)


def _node_body(h_ref, Wq_ref, Wk_ref, gv1_ref, gv2_ref, lng_ref, lnb_ref,
               q_ref, kv_ref):
    h = h_ref[...]
    mu = jnp.mean(h, axis=-1, keepdims=True)
    var = jnp.mean((h - mu) * (h - mu), axis=-1, keepdims=True)
    hn = (h - mu) * lax.rsqrt(var + 1e-5) * lng_ref[...] + lnb_ref[...]
    q_ref[...] = jnp.dot(hn, Wq_ref[...], preferred_element_type=jnp.float32)
    kv_ref[:, :D] = jnp.dot(hn, Wk_ref[...], preferred_element_type=jnp.float32)
    g = jax.nn.silu(jnp.dot(hn, gv1_ref[...], preferred_element_type=jnp.float32))
    kv_ref[:, D:] = jnp.dot(g, gv2_ref[...], preferred_element_type=jnp.float32)


def _node_call(h, Wq, Wk, gv1, gv2, lng, lnb):
    BN = 1000
    w_spec = lambda shp: pl.BlockSpec(shp, lambda i: (0, 0))
    return pl.pallas_call(
        _node_body,
        grid=(N // BN,),
        in_specs=[
            pl.BlockSpec((BN, D), lambda i: (i, 0)),
            w_spec((D, D)), w_spec((D, D)), w_spec((D, D)),
            w_spec((D, MULT * D)), w_spec((1, D)), w_spec((1, D)),
        ],
        out_specs=[
            pl.BlockSpec((BN, D), lambda i: (i, 0)),
            pl.BlockSpec((BN, D + MULT * D), lambda i: (i, 0)),
        ],
        out_shape=[
            jax.ShapeDtypeStruct((N, D), jnp.float32),
            jax.ShapeDtypeStruct((N, D + MULT * D), jnp.float32),
        ],
    )(h, Wq, Wk, gv1, gv2, lng, lnb)


def _seg_sum(x):
    return jnp.sum(x.reshape(NB, K, D), axis=1)


def _to_edges(x):
    return jnp.broadcast_to(x[:, None, :], (NB, K, D)).reshape(EB, D)


def _edge_body(t_ref, r_ref, m_ref, rl0_ref, rl1_ref, q_ref, h_ref,
               x0_ref, x1_ref, kvj_ref, xj_ref,
               Wre_ref, Wrs_ref, gtW_ref, gtb_ref, Wvq_ref, Wvk0_ref, Wvk1_ref,
               hout_ref, x0out_ref, x1out_ref, qv0_ref, qv1_ref, kvt_ref,
               gt_ref):
    t = t_ref[...]
    re = jax.nn.silu(jnp.dot(t, Wre_ref[...], preferred_element_type=jnp.float32))
    rs = jnp.dot(t, Wrs_ref[...], preferred_element_type=jnp.float32)
    kj = kvj_ref[:, :D]
    qe = _to_edges(q_ref[...])
    p = qe * kj * re
    row = lax.broadcasted_iota(jnp.int32, (D, D), 0) // DH
    col = lax.broadcasted_iota(jnp.int32, (D, D), 1) // DH
    bd = (row == col).astype(jnp.float32)
    af_pre = jnp.dot(p, bd, preferred_element_type=jnp.float32)
    r = r_ref[...]
    cut = 0.5 * (jnp.cos(r * (math.pi / CUTOFF)) + 1.0)
    cut = cut * (r < CUTOFF).astype(jnp.float32)
    scal = cut * m_ref[...] * (1.0 / 16.0)
    af = af_pre * scal

    vj = kvj_ref[:, D:]
    c0 = vj[:, 0 * D:1 * D] * rs[:, 0 * D:1 * D] * af
    c1 = vj[:, 1 * D:2 * D] * rs[:, 1 * D:2 * D] * af
    c2 = vj[:, 2 * D:3 * D] * rs[:, 2 * D:3 * D] * af
    c3 = vj[:, 3 * D:4 * D] * rs[:, 3 * D:4 * D] * af
    c4 = vj[:, 4 * D:5 * D] * rs[:, 4 * D:5 * D] * af

    hout_ref[...] = h_ref[...] + _seg_sum(c0)

    rl0 = rl0_ref[...]
    rl1 = rl1_ref[...]
    Wvq = Wvq_ref[...]
    Wvk0 = Wvk0_ref[...]
    Wvk1 = Wvk1_ref[...]
    for c in range(3):
        sl = slice(c * D, (c + 1) * D)
        dx = _seg_sum(c1 * rl0[:, c:c + 1] + c2 * xj_ref[:, sl])
        xo = x0_ref[:, sl] + dx
        x0out_ref[:, sl] = xo
        qv0_ref[:, sl] = jnp.dot(xo, Wvq, preferred_element_type=jnp.float32)
        kvt_ref[:, sl] = jnp.dot(xo, Wvk0, preferred_element_type=jnp.float32)
    for c in range(5):
        sl = slice(c * D, (c + 1) * D)
        dx = _seg_sum(c3 * rl1[:, c:c + 1] + c4 * xj_ref[:, 3 * D + c * D:3 * D + (c + 1) * D])
        xo = x1_ref[:, sl] + dx
        x1out_ref[:, sl] = xo
        qv1_ref[:, sl] = jnp.dot(xo, Wvq, preferred_element_type=jnp.float32)
        kvt_ref[:, 3 * D + c * D:3 * D + (c + 1) * D] = jnp.dot(
            xo, Wvk1, preferred_element_type=jnp.float32)

    gt_ref[...] = jax.nn.silu(
        jnp.dot(t, gtW_ref[...], preferred_element_type=jnp.float32) + gtb_ref[...])


def _edge_call(t2, r2, m2, rl0f, rl1f, q_tab, h, x0f, x1f, kvj, xj,
               Wre, Wrs, gtW, gtb, Wvq, Wvk0, Wvk1):
    eb = lambda d: pl.BlockSpec((EB, d), lambda i: (i, 0))
    nb = lambda d: pl.BlockSpec((NB, d), lambda i: (i, 0))
    w_spec = lambda shp: pl.BlockSpec(shp, lambda i: (0, 0))
    return pl.pallas_call(
        _edge_body,
        grid=(GRID,),
        in_specs=[
            eb(D), eb(1), eb(1), eb(3), eb(5),
            nb(D), nb(D), nb(3 * D), nb(5 * D),
            eb(6 * D), eb(8 * D),
            w_spec((D, D)), w_spec((D, MULT * D)), w_spec((D, D)),
            w_spec((1, D)), w_spec((D, D)), w_spec((D, D)), w_spec((D, D)),
        ],
        out_specs=[
            nb(D), nb(3 * D), nb(5 * D), nb(3 * D), nb(5 * D), nb(8 * D),
            eb(D),
        ],
        out_shape=[
            jax.ShapeDtypeStruct((N, D), jnp.float32),
            jax.ShapeDtypeStruct((N, 3 * D), jnp.float32),
            jax.ShapeDtypeStruct((N, 5 * D), jnp.float32),
            jax.ShapeDtypeStruct((N, 3 * D), jnp.float32),
            jax.ShapeDtypeStruct((N, 5 * D), jnp.float32),
            jax.ShapeDtypeStruct((N, 8 * D), jnp.float32),
            jax.ShapeDtypeStruct((E, D), jnp.float32),
        ],
    )(t2, r2, m2, rl0f, rl1f, q_tab, h, x0f, x1f, kvj, xj,
      Wre, Wrs, gtW, gtb, Wvq, Wvk0, Wvk1)


def _final_body(t_ref, gt_ref, rl0_ref, rl1_ref, qv0_ref, qv1_ref, kv2_ref,
                gwg_ref, gwW_ref, gwb_ref, tout_ref):
    rl0 = rl0_ref[...]
    rl1 = rl1_ref[...]
    acc = jnp.zeros((EB, D), jnp.float32)
    for c in range(3):
        rep = kv2_ref[:, c * D:(c + 1) * D]
        rlc = rl0[:, c:c + 1]
        rej = rep - (rlc * rlc) * jnp.sum(rep, axis=1, keepdims=True)
        acc = acc + _to_edges(qv0_ref[:, c * D:(c + 1) * D]) * rej
    for c in range(5):
        rep = kv2_ref[:, 3 * D + c * D:3 * D + (c + 1) * D]
        rlc = rl1[:, c:c + 1]
        rej = rep - (rlc * rlc) * jnp.sum(rep, axis=1, keepdims=True)
        acc = acc + _to_edges(qv1_ref[:, c * D:(c + 1) * D]) * rej
    mu = jnp.mean(acc, axis=-1, keepdims=True)
    var = jnp.mean((acc - mu) * (acc - mu), axis=-1, keepdims=True)
    an = (acc - mu) * lax.rsqrt(var + 1e-5) * gwg_ref[...]
    w = jnp.dot(jax.nn.silu(an), gwW_ref[...],
                preferred_element_type=jnp.float32) + gwb_ref[...]
    tout_ref[...] = t_ref[...] + gt_ref[...] * w


def _final_call(t2, gt, rl0f, rl1f, qv0, qv1, kv2j, gwg, gwW, gwb):
    eb = lambda d: pl.BlockSpec((EB, d), lambda i: (i, 0))
    nb = lambda d: pl.BlockSpec((NB, d), lambda i: (i, 0))
    w_spec = lambda shp: pl.BlockSpec(shp, lambda i: (0, 0))
    return pl.pallas_call(
        _final_body,
        grid=(GRID,),
        in_specs=[
            eb(D), eb(D), eb(3), eb(5), nb(3 * D), nb(5 * D), eb(8 * D),
            w_spec((1, D)), w_spec((D, D)), w_spec((1, D)),
        ],
        out_specs=eb(D),
        out_shape=jax.ShapeDtypeStruct((E, D), jnp.float32),
    )(t2, gt, rl0f, rl1f, qv0, qv1, kv2j, gwg, gwW, gwb)


def kernel(h, X_0, X_1, rl_ij_0, rl_ij_1, t_ij, r_ij, Wq, Wk, Wre, Wrs,
           gv1, gv2, Wvq, Wvk0, Wvk1, gtW, gtb, lng, lnb, gwg, gwW, gwb,
           neighbor_index, neighbor_mask):
    row = lambda v: v.reshape(1, D)
    q_tab, kv_tab = _node_call(h, Wq, Wk, gv1, gv2, row(lng), row(lnb))

    idx = neighbor_index.reshape(E).astype(jnp.int32)
    idx_pad = jnp.concatenate([idx, jnp.zeros((E_PAD - E,), jnp.int32)])

    kvj = _sc_gather_call(kv_tab, idx_pad, ch=80)
    x_cat = jnp.concatenate([X_0.reshape(N, 3 * D), X_1.reshape(N, 5 * D)],
                            axis=1)
    xj = _sc_gather_call(x_cat, idx_pad, ch=40)

    t2 = t_ij.reshape(E, D)
    r2 = r_ij.reshape(E, 1)
    m2 = neighbor_mask.reshape(E, 1).astype(jnp.float32)
    rl0f = rl_ij_0.reshape(E, 3)
    rl1f = rl_ij_1.reshape(E, 5)

    (hout, x0out, x1out, qv0, qv1, kv2_tab, gt) = _edge_call(
        t2, r2, m2, rl0f, rl1f, q_tab, h,
        X_0.reshape(N, 3 * D), X_1.reshape(N, 5 * D), kvj, xj,
        Wre, Wrs, gtW, row(gtb), Wvq, Wvk0, Wvk1)

    kv2j = _sc_gather_call(kv2_tab, idx_pad, ch=40)

    tout = _final_call(t2, gt, rl0f, rl1f, qv0, qv1, kv2j,
                       row(gwg), gwW, row(gwb))

    return (hout, x0out.reshape(N, 3, D), x1out.reshape(N, 5, D),
            tout.reshape(N, K, D))
